# baseline (speedup 1.0000x reference)
import jax
import jax.numpy as jnp
from jax.experimental import pallas as pl
from jax.experimental.pallas import tpu as pltpu

PAD, UNK, END = 0, 1, 2

B, L, V = 16, 512, 32000
ROWS_PER_CORE = 8
NCHUNK = L // 8
NSLOT = ROWS_PER_CORE * L + ROWS_PER_CORE
NSEM = 64

XB_W = 768
BITS0, POPS0, NCH0 = 512, 576, 640


def _ti_loss_kernel(seq_sm, ins_sm, fwd_ref, tgt_ref, logits_ref, out_ref,
                    slab, msub, mlane, mrow, xb_vmem, xb_sm, sems, sem2):
    core = pl.program_id(0)
    base_b = core * ROWS_PER_CORE

    for j in range(ROWS_PER_CORE):
        b = base_b + j
        s2 = seq_sm[b] + 2
        l8 = pl.multiple_of((s2 >> 3) << 3, 8)
        pltpu.make_async_copy(
            logits_ref.at[pl.ds(b, 1), pl.ds(l8, 8), pl.ds(0, 128)],
            slab.at[pl.ds(j, 1)], sems.at[j]).start()
        msub[j] = s2 & 7

    fwdv = fwd_ref[...]
    tgtv = tgt_ref[...]
    liota = jax.lax.broadcasted_iota(jnp.int32, (ROWS_PER_CORE, L), 1)
    fp = jnp.min(jnp.where(fwdv == PAD, liota, L), axis=1,
                 keepdims=True)
    unk = (liota < fp) & (fwdv == UNK)
    bitsf = unk.astype(jnp.float32)

    riota = jax.lax.broadcasted_iota(jnp.int32, (L, 128), 0)
    ciota = jax.lax.broadcasted_iota(jnp.int32, (L, 128), 1)
    r3 = riota >> 3
    w = jnp.exp2((riota & 7).astype(jnp.float32))
    pack_m = (jnp.where(r3 == ciota, w, 0.0)
              + jnp.where(r3 == ciota - 64, 1.0, 0.0))
    cmb = jnp.dot(bitsf, pack_m,
                  preferred_element_type=jnp.float32).astype(jnp.int32)

    nch = jnp.minimum((fp >> 3) + 1, NCHUNK)
    xb_vmem[:, 0:512] = jnp.roll(tgtv, 1, axis=1)
    xb_vmem[:, 512:640] = cmb
    xb_vmem[:, 640:768] = jnp.broadcast_to(nch, (ROWS_PER_CORE, 128))
    pltpu.make_async_copy(xb_vmem, xb_sm, sem2).start()
    pltpu.make_async_copy(xb_vmem, xb_sm, sem2).wait()

    cnt = jnp.int32(ROWS_PER_CORE)
    for j in range(ROWS_PER_CORE):
        def chunk_body(k, cnt, j=j):
            bits = xb_sm[j, BITS0 + k]

            @pl.when(bits != 0)
            def _chunk():
                b = base_b + j
                tl = pl.multiple_of(k << 3, 8)
                slot = cnt
                for i in range(8):
                    hit = ((bits >> i) & 1) == 1

                    def _issue(i=i, slot=slot, b=b, tl=tl):
                        t = xb_sm[j, tl + i]
                        cb = pl.multiple_of((t >> 7) << 7, 128)
                        pltpu.make_async_copy(
                            logits_ref.at[pl.ds(b, 1), pl.ds(tl, 8),
                                          pl.ds(cb, 128)],
                            slab.at[pl.ds(slot, 1)],
                            sems.at[jnp.minimum(slot, NSEM)]).start()
                        msub[slot] = i
                        mlane[slot] = t & 127
                        mrow[slot] = j

                    pl.when(hit)(_issue)
                    slot = slot + hit.astype(jnp.int32)

            return cnt + xb_sm[j, POPS0 + k]

        cnt = jax.lax.fori_loop(0, xb_sm[j, NCH0], chunk_body, cnt)
    cnt_fin = cnt

    sub_iota = jax.lax.broadcasted_iota(jnp.int32, (8, 128), 0)
    lane_iota = jax.lax.broadcasted_iota(jnp.int32, (8, 128), 1)
    row_iota = jax.lax.broadcasted_iota(jnp.int32, (8, 1), 0)

    fbmat = jnp.zeros((8, 128), jnp.float32)
    for j in range(ROWS_PER_CORE):
        pltpu.make_async_copy(
            slab.at[pl.ds(0, 1)], slab.at[pl.ds(0, 1)], sems.at[j]).wait()
        chunk = slab[j]
        rv = jnp.sum(jnp.where(sub_iota == msub[j], chunk, 0.0),
                     axis=0, keepdims=True)
        fbmat = fbmat + jnp.where(row_iota == j, rv, 0.0)
    fbp = fbmat[:, END:END + 1]

    def ext_core(k, carry):
        acc, cntm = carry
        chunk = slab[k]
        rv = jnp.sum(jnp.where(sub_iota == msub[k], chunk, 0.0),
                     axis=0, keepdims=True)
        lm = lane_iota[0:1, :] == mlane[k]
        rowmask = row_iota == mrow[k]
        hit = rowmask & lm
        nll = -jnp.log(rv)
        acc = acc + jnp.where(hit, nll, 0.0)
        cntm = cntm + jnp.where(hit, 1.0, 0.0)
        return acc, cntm

    def ext_wait_body(k, carry):
        pltpu.make_async_copy(
            slab.at[pl.ds(0, 1)], slab.at[pl.ds(0, 1)], sems.at[k]).wait()
        return ext_core(k, carry)

    acc0 = jnp.zeros((8, 128), jnp.float32)
    acc, cntm = jax.lax.fori_loop(
        ROWS_PER_CORE, jnp.minimum(cnt_fin, NSEM), ext_wait_body,
        (acc0, acc0))

    def ovf_wait_body(_, carry):
        pltpu.make_async_copy(
            slab.at[pl.ds(0, 1)], slab.at[pl.ds(0, 1)],
            sems.at[NSEM]).wait()
        return carry

    jax.lax.fori_loop(NSEM, cnt_fin, ovf_wait_body, 0)
    acc, cntm = jax.lax.fori_loop(NSEM, cnt_fin, ext_core, (acc, cntm))

    ssum = jnp.sum(acc, axis=1, keepdims=True)
    cnt_v = jnp.sum(cntm, axis=1, keepdims=True)
    smean = ssum / jnp.maximum(cnt_v, 1.0)
    sent = jnp.where(cnt_v > 0, smean, -jnp.log(fbp))

    active = jnp.zeros((8, 1), jnp.float32)
    for j in range(ROWS_PER_CORE):
        b = base_b + j
        a = (ins_sm[b] < seq_sm[b]).astype(jnp.float32)
        active = active + jnp.where(row_iota == j, a, 0.0)

    num = jnp.sum(sent * active)
    den = jnp.sum(active)
    li = jax.lax.broadcasted_iota(jnp.int32, (1, 128), 1)
    out_ref[0] = jnp.where(li == 0, num, jnp.where(li == 1, den, 0.0))


def kernel(logits, forwarded_trgs, targets, sequence_lengths, inserted):
    fwd = forwarded_trgs.astype(jnp.int32)
    tgt = targets.astype(jnp.int32)
    seq = sequence_lengths.astype(jnp.int32)
    ins = inserted.astype(jnp.int32)

    out = pl.pallas_call(
        _ti_loss_kernel,
        grid_spec=pltpu.PrefetchScalarGridSpec(
            num_scalar_prefetch=2,
            grid=(2,),
            in_specs=[
                pl.BlockSpec((ROWS_PER_CORE, L), lambda i, *_: (i, 0)),
                pl.BlockSpec((ROWS_PER_CORE, L), lambda i, *_: (i, 0)),
                pl.BlockSpec(memory_space=pl.ANY),
            ],
            out_specs=pl.BlockSpec((1, 1, 128), lambda i, *_: (i, 0, 0)),
            scratch_shapes=[
                pltpu.VMEM((NSLOT, 8, 128), jnp.float32),
                pltpu.SMEM((NSLOT,), jnp.int32),
                pltpu.SMEM((NSLOT,), jnp.int32),
                pltpu.SMEM((NSLOT,), jnp.int32),
                pltpu.VMEM((ROWS_PER_CORE, XB_W), jnp.int32),
                pltpu.SMEM((ROWS_PER_CORE, XB_W), jnp.int32),
                pltpu.SemaphoreType.DMA((NSEM + 1,)),
                pltpu.SemaphoreType.DMA,
            ],
        ),
        out_shape=jax.ShapeDtypeStruct((2, 1, 128), jnp.float32),
        compiler_params=pltpu.CompilerParams(
            dimension_semantics=("parallel",),
            vmem_limit_bytes=56 * 1024 * 1024,
        ),
    )(seq, ins, fwd, tgt, logits)

    num = out[0, 0, 0] + out[1, 0, 0]
    den = jnp.maximum(out[0, 0, 1] + out[1, 0, 1], 1.0)
    return num / den


# speedup vs baseline: 1.0250x; 1.0250x over previous
import jax
import jax.numpy as jnp
from jax.experimental import pallas as pl
from jax.experimental.pallas import tpu as pltpu

PAD, UNK, END = 0, 1, 2

B, L, V = 16, 512, 32000
ROWS_PER_CORE = 8
NCHUNK = L // 8
NSLOT = ROWS_PER_CORE * L + ROWS_PER_CORE

XB_W = 768
BITS0, POPS0, NCH0 = 512, 576, 640


def _ti_loss_kernel(seq_sm, ins_sm, fwd_ref, tgt_ref, logits_ref, out_ref,
                    slab, msub, xb_vmem, xb_sm, sem, sem2):
    core = pl.program_id(0)
    base_b = core * ROWS_PER_CORE

    for j in range(ROWS_PER_CORE):
        b = base_b + j
        s2 = seq_sm[b] + 2
        l8 = pl.multiple_of((s2 >> 3) << 3, 8)
        pltpu.make_async_copy(
            logits_ref.at[pl.ds(b, 1), pl.ds(l8, 8), pl.ds(0, 128)],
            slab.at[pl.ds(j, 1)], sem).start()
        msub[j] = (s2 & 7) | (END << 3) | (j << 10)

    fwdv = fwd_ref[...]
    tgtv = tgt_ref[...]
    liota = jax.lax.broadcasted_iota(jnp.int32, (ROWS_PER_CORE, L), 1)
    fp = jnp.min(jnp.where(fwdv == PAD, liota, L), axis=1,
                 keepdims=True)
    unk = (liota < fp) & (fwdv == UNK)
    bitsf = unk.astype(jnp.float32)

    riota = jax.lax.broadcasted_iota(jnp.int32, (L, 128), 0)
    ciota = jax.lax.broadcasted_iota(jnp.int32, (L, 128), 1)
    r3 = riota >> 3
    w = jnp.exp2((riota & 7).astype(jnp.float32))
    pack_m = (jnp.where(r3 == ciota, w, 0.0)
              + jnp.where(r3 == ciota - 64, 1.0, 0.0))
    cmb = jnp.dot(bitsf, pack_m,
                  preferred_element_type=jnp.float32).astype(jnp.int32)

    nch = jnp.minimum((fp >> 3) + 1, NCHUNK)
    xb_vmem[:, 0:512] = jnp.roll(tgtv, 1, axis=1)
    xb_vmem[:, 512:640] = cmb
    xb_vmem[:, 640:768] = jnp.broadcast_to(nch, (ROWS_PER_CORE, 128))
    pltpu.make_async_copy(xb_vmem, xb_sm, sem2).start()

    pltpu.make_async_copy(xb_vmem, xb_sm, sem2).wait()

    cnt = jnp.int32(ROWS_PER_CORE)
    for j in range(ROWS_PER_CORE):
        def chunk_body(k, cnt, j=j):
            bits = xb_sm[j, BITS0 + k]

            @pl.when(bits != 0)
            def _chunk():
                b = base_b + j
                tl = pl.multiple_of(k << 3, 8)
                slot = cnt
                for i in range(8):
                    hit = ((bits >> i) & 1) == 1

                    def _issue(i=i, slot=slot, b=b, tl=tl):
                        t = xb_sm[j, tl + i]
                        cb = pl.multiple_of((t >> 7) << 7, 128)
                        pltpu.make_async_copy(
                            logits_ref.at[pl.ds(b, 1), pl.ds(tl, 8),
                                          pl.ds(cb, 128)],
                            slab.at[pl.ds(slot, 1)], sem).start()
                        msub[slot] = i | ((t & 127) << 3) | (j << 10)

                    pl.when(hit)(_issue)
                    slot = slot + hit.astype(jnp.int32)

            return cnt + xb_sm[j, POPS0 + k]

        cnt = jax.lax.fori_loop(0, xb_sm[j, NCH0], chunk_body, cnt)
    cnt_fin = cnt

    def wait8_body(_, carry):
        pltpu.make_async_copy(
            slab.at[pl.ds(0, 8)], slab.at[pl.ds(0, 8)], sem).wait()
        return carry

    def wait1_body(_, carry):
        pltpu.make_async_copy(
            slab.at[pl.ds(0, 1)], slab.at[pl.ds(0, 1)], sem).wait()
        return carry

    jax.lax.fori_loop(0, cnt_fin >> 3, wait8_body, 0)
    jax.lax.fori_loop(0, cnt_fin & 7, wait1_body, 0)

    sub_iota = jax.lax.broadcasted_iota(jnp.int32, (8, 128), 0)
    lane_iota = jax.lax.broadcasted_iota(jnp.int32, (8, 128), 1)
    row_iota = jax.lax.broadcasted_iota(jnp.int32, (8, 1), 0)

    fbmat = jnp.zeros((8, 128), jnp.float32)
    for j in range(ROWS_PER_CORE):
        chunk = slab[j]
        rv = jnp.sum(jnp.where(sub_iota == (msub[j] & 7), chunk, 0.0),
                     axis=0, keepdims=True)
        fbmat = fbmat + jnp.where(row_iota == j, rv, 0.0)
    fbp = fbmat[:, END:END + 1]

    def ext_body(k, carry):
        acc, cntm = carry
        chunk = slab[k]
        m = msub[k]
        rv = jnp.sum(jnp.where(sub_iota == (m & 7), chunk, 0.0),
                     axis=0, keepdims=True)
        lm = lane_iota[0:1, :] == ((m >> 3) & 127)
        rowmask = row_iota == (m >> 10)
        hit = rowmask & lm
        nll = -jnp.log(rv)
        acc = acc + jnp.where(hit, nll, 0.0)
        cntm = cntm + jnp.where(hit, 1.0, 0.0)
        return acc, cntm

    acc0 = jnp.zeros((8, 128), jnp.float32)
    acc, cntm = jax.lax.fori_loop(ROWS_PER_CORE, cnt_fin, ext_body,
                                  (acc0, acc0))

    ssum = jnp.sum(acc, axis=1, keepdims=True)
    cnt_v = jnp.sum(cntm, axis=1, keepdims=True)
    smean = ssum / jnp.maximum(cnt_v, 1.0)
    sent = jnp.where(cnt_v > 0, smean, -jnp.log(fbp))

    active = jnp.zeros((8, 1), jnp.float32)
    for j in range(ROWS_PER_CORE):
        b = base_b + j
        a = (ins_sm[b] < seq_sm[b]).astype(jnp.float32)
        active = active + jnp.where(row_iota == j, a, 0.0)

    num = jnp.sum(sent * active)
    den = jnp.sum(active)
    li = jax.lax.broadcasted_iota(jnp.int32, (1, 128), 1)
    out_ref[0] = jnp.where(li == 0, num, jnp.where(li == 1, den, 0.0))


def kernel(logits, forwarded_trgs, targets, sequence_lengths, inserted):
    fwd = forwarded_trgs.astype(jnp.int32)
    tgt = targets.astype(jnp.int32)
    seq = sequence_lengths.astype(jnp.int32)
    ins = inserted.astype(jnp.int32)

    out = pl.pallas_call(
        _ti_loss_kernel,
        grid_spec=pltpu.PrefetchScalarGridSpec(
            num_scalar_prefetch=2,
            grid=(2,),
            in_specs=[
                pl.BlockSpec((ROWS_PER_CORE, L), lambda i, *_: (i, 0)),
                pl.BlockSpec((ROWS_PER_CORE, L), lambda i, *_: (i, 0)),
                pl.BlockSpec(memory_space=pl.ANY),
            ],
            out_specs=pl.BlockSpec((1, 1, 128), lambda i, *_: (i, 0, 0)),
            scratch_shapes=[
                pltpu.VMEM((NSLOT, 8, 128), jnp.float32),
                pltpu.SMEM((NSLOT,), jnp.int32),
                pltpu.VMEM((ROWS_PER_CORE, XB_W), jnp.int32),
                pltpu.SMEM((ROWS_PER_CORE, XB_W), jnp.int32),
                pltpu.SemaphoreType.DMA,
                pltpu.SemaphoreType.DMA,
            ],
        ),
        out_shape=jax.ShapeDtypeStruct((2, 1, 128), jnp.float32),
        compiler_params=pltpu.CompilerParams(
            dimension_semantics=("parallel",),
            vmem_limit_bytes=56 * 1024 * 1024,
        ),
    )(seq, ins, fwd, tgt, logits)

    num = out[0, 0, 0] + out[1, 0, 0]
    den = jnp.maximum(out[0, 0, 1] + out[1, 0, 1], 1.0)
    return num / den


# speedup vs baseline: 1.0593x; 1.0335x over previous
import jax
import jax.numpy as jnp
from jax.experimental import pallas as pl
from jax.experimental.pallas import tpu as pltpu

PAD, UNK, END = 0, 1, 2

B, L, V = 16, 512, 32000
ROWS_PER_CORE = 8
NCHUNK = L // 8
NSLOT = ROWS_PER_CORE * L + ROWS_PER_CORE

XB_W = 256


def _ti_loss_kernel(seq_sm, ins_sm, fwd_ref, tgt_ref, logits_ref, out_ref,
                    slab, msub, xa_vmem, xa_sm, xb_vmem, xb_sm, sem, sem2, sem3):
    core = pl.program_id(0)
    base_b = core * ROWS_PER_CORE

    for j in range(ROWS_PER_CORE):
        b = base_b + j
        s2 = seq_sm[b] + 2
        l8 = pl.multiple_of((s2 >> 3) << 3, 8)
        pltpu.make_async_copy(
            logits_ref.at[pl.ds(b, 1), pl.ds(l8, 8), pl.ds(0, 128)],
            slab.at[pl.ds(j, 1)], sem).start()
        msub[j] = (s2 & 7) | (END << 3) | (j << 10)

    tgtv = tgt_ref[...]
    xa_vmem[...] = jnp.roll(tgtv, 1, axis=1)
    pltpu.make_async_copy(xa_vmem, xa_sm, sem2).start()

    fwdv = fwd_ref[...]
    liota = jax.lax.broadcasted_iota(jnp.int32, (ROWS_PER_CORE, L), 1)
    fp = jnp.min(jnp.where(fwdv == PAD, liota, L), axis=1,
                 keepdims=True)
    unk = (liota < fp) & (fwdv == UNK)
    bitsf = unk.astype(jnp.float32)

    riota = jax.lax.broadcasted_iota(jnp.int32, (L, 128), 0)
    ciota = jax.lax.broadcasted_iota(jnp.int32, (L, 128), 1)
    r3 = riota >> 3
    w = jnp.exp2((riota & 7).astype(jnp.float32))
    pack_m = (jnp.where(r3 == ciota, w, 0.0)
              + jnp.where(r3 == ciota - 64, 1.0, 0.0))
    cmb = jnp.dot(bitsf, pack_m,
                  preferred_element_type=jnp.float32).astype(jnp.int32)

    nch = jnp.minimum((fp >> 3) + 1, NCHUNK)
    xb_vmem[:, 0:128] = cmb
    xb_vmem[:, 128:256] = jnp.broadcast_to(nch, (ROWS_PER_CORE, 128))
    pltpu.make_async_copy(xb_vmem, xb_sm, sem3).start()

    pltpu.make_async_copy(xa_vmem, xa_sm, sem2).wait()
    pltpu.make_async_copy(xb_vmem, xb_sm, sem3).wait()

    cnt = jnp.int32(ROWS_PER_CORE)
    for j in range(ROWS_PER_CORE):
        def chunk_body(k, cnt, j=j):
            bits = xb_sm[j, k]

            @pl.when(bits != 0)
            def _chunk():
                b = base_b + j
                tl = pl.multiple_of(k << 3, 8)
                slot = cnt
                for i in range(8):
                    hit = ((bits >> i) & 1) == 1

                    def _issue(i=i, slot=slot, b=b, tl=tl):
                        t = xa_sm[j, tl + i]
                        cb = pl.multiple_of((t >> 7) << 7, 128)
                        pltpu.make_async_copy(
                            logits_ref.at[pl.ds(b, 1), pl.ds(tl, 8),
                                          pl.ds(cb, 128)],
                            slab.at[pl.ds(slot, 1)], sem).start()
                        msub[slot] = i | ((t & 127) << 3) | (j << 10)

                    pl.when(hit)(_issue)
                    slot = slot + hit.astype(jnp.int32)

            return cnt + xb_sm[j, 64 + k]

        cnt = jax.lax.fori_loop(0, xb_sm[j, 128], chunk_body, cnt)
    cnt_fin = cnt

    def wait8_body(_, carry):
        pltpu.make_async_copy(
            slab.at[pl.ds(0, 8)], slab.at[pl.ds(0, 8)], sem).wait()
        return carry

    def wait1_body(_, carry):
        pltpu.make_async_copy(
            slab.at[pl.ds(0, 1)], slab.at[pl.ds(0, 1)], sem).wait()
        return carry

    jax.lax.fori_loop(0, cnt_fin >> 3, wait8_body, 0)
    jax.lax.fori_loop(0, cnt_fin & 7, wait1_body, 0)

    sub_iota = jax.lax.broadcasted_iota(jnp.int32, (8, 128), 0)
    lane_iota = jax.lax.broadcasted_iota(jnp.int32, (8, 128), 1)
    row_iota = jax.lax.broadcasted_iota(jnp.int32, (8, 1), 0)

    fbmat = jnp.zeros((8, 128), jnp.float32)
    for j in range(ROWS_PER_CORE):
        chunk = slab[j]
        rv = jnp.sum(jnp.where(sub_iota == (msub[j] & 7), chunk, 0.0),
                     axis=0, keepdims=True)
        fbmat = fbmat + jnp.where(row_iota == j, rv, 0.0)
    fbp = fbmat[:, END:END + 1]

    def ext_body(k, carry):
        acc, cntm = carry
        chunk = slab[k]
        m = msub[k]
        rv = jnp.sum(jnp.where(sub_iota == (m & 7), chunk, 0.0),
                     axis=0, keepdims=True)
        lm = lane_iota[0:1, :] == ((m >> 3) & 127)
        rowmask = row_iota == (m >> 10)
        hit = rowmask & lm
        nll = -jnp.log(rv)
        acc = acc + jnp.where(hit, nll, 0.0)
        cntm = cntm + jnp.where(hit, 1.0, 0.0)
        return acc, cntm

    def ext_pair(p, carry):
        k = ROWS_PER_CORE + p * 2
        return ext_body(k + 1, ext_body(k, carry))

    acc0 = jnp.zeros((8, 128), jnp.float32)
    npairs = (cnt_fin - ROWS_PER_CORE) >> 1
    acc, cntm = jax.lax.fori_loop(0, npairs, ext_pair, (acc0, acc0))
    acc, cntm = jax.lax.fori_loop(ROWS_PER_CORE + npairs * 2, cnt_fin,
                                  ext_body, (acc, cntm))

    ssum = jnp.sum(acc, axis=1, keepdims=True)
    cnt_v = jnp.sum(cntm, axis=1, keepdims=True)
    smean = ssum / jnp.maximum(cnt_v, 1.0)
    sent = jnp.where(cnt_v > 0, smean, -jnp.log(fbp))

    active = jnp.zeros((8, 1), jnp.float32)
    for j in range(ROWS_PER_CORE):
        b = base_b + j
        a = (ins_sm[b] < seq_sm[b]).astype(jnp.float32)
        active = active + jnp.where(row_iota == j, a, 0.0)

    num = jnp.sum(sent * active)
    den = jnp.sum(active)
    li = jax.lax.broadcasted_iota(jnp.int32, (1, 128), 1)
    out_ref[0] = jnp.where(li == 0, num, jnp.where(li == 1, den, 0.0))


def kernel(logits, forwarded_trgs, targets, sequence_lengths, inserted):
    fwd = forwarded_trgs.astype(jnp.int32)
    tgt = targets.astype(jnp.int32)
    seq = sequence_lengths.astype(jnp.int32)
    ins = inserted.astype(jnp.int32)

    out = pl.pallas_call(
        _ti_loss_kernel,
        grid_spec=pltpu.PrefetchScalarGridSpec(
            num_scalar_prefetch=2,
            grid=(2,),
            in_specs=[
                pl.BlockSpec((ROWS_PER_CORE, L), lambda i, *_: (i, 0)),
                pl.BlockSpec((ROWS_PER_CORE, L), lambda i, *_: (i, 0)),
                pl.BlockSpec(memory_space=pl.ANY),
            ],
            out_specs=pl.BlockSpec((1, 1, 128), lambda i, *_: (i, 0, 0)),
            scratch_shapes=[
                pltpu.VMEM((NSLOT, 8, 128), jnp.float32),
                pltpu.SMEM((NSLOT,), jnp.int32),
                pltpu.VMEM((ROWS_PER_CORE, L), jnp.int32),
                pltpu.SMEM((ROWS_PER_CORE, L), jnp.int32),
                pltpu.VMEM((ROWS_PER_CORE, XB_W), jnp.int32),
                pltpu.SMEM((ROWS_PER_CORE, XB_W), jnp.int32),
                pltpu.SemaphoreType.DMA,
                pltpu.SemaphoreType.DMA,
                pltpu.SemaphoreType.DMA,
            ],
        ),
        out_shape=jax.ShapeDtypeStruct((2, 1, 128), jnp.float32),
        compiler_params=pltpu.CompilerParams(
            dimension_semantics=("parallel",),
            vmem_limit_bytes=56 * 1024 * 1024,
        ),
    )(seq, ins, fwd, tgt, logits)

    num = out[0, 0, 0] + out[1, 0, 0]
    den = jnp.maximum(out[0, 0, 1] + out[1, 0, 1], 1.0)
    return num / den


# speedup vs baseline: 1.1273x; 1.0642x over previous
import jax
import jax.numpy as jnp
from jax.experimental import pallas as pl
from jax.experimental.pallas import tpu as pltpu

PAD, UNK, END = 0, 1, 2

B, L, V = 16, 512, 32000
ROWS_PER_CORE = 8
NCHUNK = L // 8
NSLOT = ROWS_PER_CORE * L + ROWS_PER_CORE

XB_W = 256


def _ti_loss_kernel(seq_sm, ins_sm, fwd_ref, tgt_ref, logits_ref, out_ref,
                    slab, msub, xa_vmem, xa_sm, xb_vmem, xb_sm, sem, sem2, sem3):
    core = pl.program_id(0)
    base_b = core * ROWS_PER_CORE

    for j in range(ROWS_PER_CORE):
        b = base_b + j
        s2 = seq_sm[b] + 2
        l8 = pl.multiple_of((s2 >> 3) << 3, 8)
        pltpu.make_async_copy(
            logits_ref.at[pl.ds(b, 1), pl.ds(l8, 8), pl.ds(0, 128)],
            slab.at[pl.ds(j, 1)], sem).start()
        msub[j] = (s2 & 7) | (END << 3) | (j << 10)

    tgtv = tgt_ref[...]
    xa_vmem[...] = jnp.roll(tgtv, 1, axis=1)
    pltpu.make_async_copy(xa_vmem, xa_sm, sem2).start()

    fwdv = fwd_ref[...]
    liota = jax.lax.broadcasted_iota(jnp.int32, (ROWS_PER_CORE, L), 1)
    fp = jnp.min(jnp.where(fwdv == PAD, liota, L), axis=1,
                 keepdims=True)
    unk = (liota < fp) & (fwdv == UNK)
    bitsf = unk.astype(jnp.float32)

    riota = jax.lax.broadcasted_iota(jnp.int32, (L, 128), 0)
    ciota = jax.lax.broadcasted_iota(jnp.int32, (L, 128), 1)
    r3 = riota >> 3
    w = jnp.exp2((riota & 7).astype(jnp.float32))
    pack_m = (jnp.where(r3 == ciota, w, 0.0)
              + jnp.where(r3 == ciota - 64, 1.0, 0.0))
    cmb = jnp.dot(bitsf, pack_m,
                  preferred_element_type=jnp.float32).astype(jnp.int32)

    nch = jnp.minimum((fp >> 3) + 1, NCHUNK)
    xb_vmem[:, 0:128] = cmb
    xb_vmem[:, 128:256] = jnp.broadcast_to(nch, (ROWS_PER_CORE, 128))
    pltpu.make_async_copy(xb_vmem, xb_sm, sem3).start()

    pltpu.make_async_copy(xa_vmem, xa_sm, sem2).wait()
    pltpu.make_async_copy(xb_vmem, xb_sm, sem3).wait()

    cnt = jnp.int32(ROWS_PER_CORE)
    for j in range(ROWS_PER_CORE):
        def chunk_body(k, cnt, j=j):
            bits = xb_sm[j, k]
            pops = xb_sm[j, 64 + k]

            @pl.when(bits != 0)
            def _chunk():
                b = base_b + j
                tl = pl.multiple_of(k << 3, 8)

                def hit_body(_, st, b=b, tl=tl, j=j):
                    bits_c, slot = st
                    low = bits_c & -bits_c
                    i = (((low & 0xAA) != 0).astype(jnp.int32)
                         + 2 * ((low & 0xCC) != 0).astype(jnp.int32)
                         + 4 * ((low & 0xF0) != 0).astype(jnp.int32))
                    t = xa_sm[j, tl + i]
                    cb = pl.multiple_of((t >> 7) << 7, 128)
                    pltpu.make_async_copy(
                        logits_ref.at[pl.ds(b, 1), pl.ds(tl, 8),
                                      pl.ds(cb, 128)],
                        slab.at[pl.ds(slot, 1)], sem).start()
                    msub[slot] = i | ((t & 127) << 3) | (j << 10)
                    return bits_c ^ low, slot + 1

                jax.lax.fori_loop(0, pops, hit_body, (bits, cnt))

            return cnt + pops

        cnt = jax.lax.fori_loop(0, xb_sm[j, 128], chunk_body, cnt)
    cnt_fin = cnt

    def wait8_body(_, carry):
        pltpu.make_async_copy(
            slab.at[pl.ds(0, 8)], slab.at[pl.ds(0, 8)], sem).wait()
        return carry

    def wait1_body(_, carry):
        pltpu.make_async_copy(
            slab.at[pl.ds(0, 1)], slab.at[pl.ds(0, 1)], sem).wait()
        return carry

    jax.lax.fori_loop(0, cnt_fin >> 3, wait8_body, 0)
    jax.lax.fori_loop(0, cnt_fin & 7, wait1_body, 0)

    sub_iota = jax.lax.broadcasted_iota(jnp.int32, (8, 128), 0)
    lane_iota = jax.lax.broadcasted_iota(jnp.int32, (8, 128), 1)
    row_iota = jax.lax.broadcasted_iota(jnp.int32, (8, 1), 0)

    fbmat = jnp.zeros((8, 128), jnp.float32)
    for j in range(ROWS_PER_CORE):
        chunk = slab[j]
        rv = jnp.sum(jnp.where(sub_iota == (msub[j] & 7), chunk, 0.0),
                     axis=0, keepdims=True)
        fbmat = fbmat + jnp.where(row_iota == j, rv, 0.0)
    fbp = fbmat[:, END:END + 1]

    def ext_body(k, carry):
        acc, cntm = carry
        chunk = slab[k]
        m = msub[k]
        rv = jnp.sum(jnp.where(sub_iota == (m & 7), chunk, 0.0),
                     axis=0, keepdims=True)
        lm = lane_iota[0:1, :] == ((m >> 3) & 127)
        rowmask = row_iota == (m >> 10)
        hit = rowmask & lm
        nll = -jnp.log(rv)
        acc = acc + jnp.where(hit, nll, 0.0)
        cntm = cntm + jnp.where(hit, 1.0, 0.0)
        return acc, cntm

    def ext_pair(p, carry):
        k = ROWS_PER_CORE + p * 2
        return ext_body(k + 1, ext_body(k, carry))

    acc0 = jnp.zeros((8, 128), jnp.float32)
    npairs = (cnt_fin - ROWS_PER_CORE) >> 1
    acc, cntm = jax.lax.fori_loop(0, npairs, ext_pair, (acc0, acc0))
    acc, cntm = jax.lax.fori_loop(ROWS_PER_CORE + npairs * 2, cnt_fin,
                                  ext_body, (acc, cntm))

    ssum = jnp.sum(acc, axis=1, keepdims=True)
    cnt_v = jnp.sum(cntm, axis=1, keepdims=True)
    smean = ssum / jnp.maximum(cnt_v, 1.0)
    sent = jnp.where(cnt_v > 0, smean, -jnp.log(fbp))

    active = jnp.zeros((8, 1), jnp.float32)
    for j in range(ROWS_PER_CORE):
        b = base_b + j
        a = (ins_sm[b] < seq_sm[b]).astype(jnp.float32)
        active = active + jnp.where(row_iota == j, a, 0.0)

    num = jnp.sum(sent * active)
    den = jnp.sum(active)
    li = jax.lax.broadcasted_iota(jnp.int32, (1, 128), 1)
    out_ref[0] = jnp.where(li == 0, num, jnp.where(li == 1, den, 0.0))


def kernel(logits, forwarded_trgs, targets, sequence_lengths, inserted):
    fwd = forwarded_trgs.astype(jnp.int32)
    tgt = targets.astype(jnp.int32)
    seq = sequence_lengths.astype(jnp.int32)
    ins = inserted.astype(jnp.int32)

    out = pl.pallas_call(
        _ti_loss_kernel,
        grid_spec=pltpu.PrefetchScalarGridSpec(
            num_scalar_prefetch=2,
            grid=(2,),
            in_specs=[
                pl.BlockSpec((ROWS_PER_CORE, L), lambda i, *_: (i, 0)),
                pl.BlockSpec((ROWS_PER_CORE, L), lambda i, *_: (i, 0)),
                pl.BlockSpec(memory_space=pl.ANY),
            ],
            out_specs=pl.BlockSpec((1, 1, 128), lambda i, *_: (i, 0, 0)),
            scratch_shapes=[
                pltpu.VMEM((NSLOT, 8, 128), jnp.float32),
                pltpu.SMEM((NSLOT,), jnp.int32),
                pltpu.VMEM((ROWS_PER_CORE, L), jnp.int32),
                pltpu.SMEM((ROWS_PER_CORE, L), jnp.int32),
                pltpu.VMEM((ROWS_PER_CORE, XB_W), jnp.int32),
                pltpu.SMEM((ROWS_PER_CORE, XB_W), jnp.int32),
                pltpu.SemaphoreType.DMA,
                pltpu.SemaphoreType.DMA,
                pltpu.SemaphoreType.DMA,
            ],
        ),
        out_shape=jax.ShapeDtypeStruct((2, 1, 128), jnp.float32),
        compiler_params=pltpu.CompilerParams(
            dimension_semantics=("parallel",),
            vmem_limit_bytes=56 * 1024 * 1024,
        ),
    )(seq, ins, fwd, tgt, logits)

    num = out[0, 0, 0] + out[1, 0, 0]
    den = jnp.maximum(out[0, 0, 1] + out[1, 0, 1], 1.0)
    return num / den


# speedup vs baseline: 1.1273x; 1.0000x over previous
import jax
import jax.numpy as jnp
from jax.experimental import pallas as pl
from jax.experimental.pallas import tpu as pltpu

PAD, UNK, END = 0, 1, 2

B, L, V = 16, 512, 32000
ROWS_PER_CORE = 8
NCHUNK = L // 8
NSLOT = ROWS_PER_CORE * L + ROWS_PER_CORE

XB_W = 256


def _ti_loss_kernel(seq_sm, ins_sm, fwd_ref, tgt_ref, logits_ref, out_ref,
                    slab, msub, xa_vmem, xa_sm, xb_vmem, xb_sm, sem, sem2, sem3):
    core = pl.program_id(0)
    base_b = core * ROWS_PER_CORE

    for j in range(ROWS_PER_CORE):
        b = base_b + j
        s2 = seq_sm[b] + 2
        l8 = pl.multiple_of((s2 >> 3) << 3, 8)
        pltpu.make_async_copy(
            logits_ref.at[pl.ds(b, 1), pl.ds(l8, 8), pl.ds(0, 128)],
            slab.at[pl.ds(j, 1)], sem).start()
        msub[j] = (s2 & 7) | (END << 3) | (j << 10)

    tgtv = tgt_ref[...]
    xa_vmem[...] = jnp.roll(tgtv, 1, axis=1)
    pltpu.make_async_copy(xa_vmem, xa_sm, sem2).start()

    fwdv = fwd_ref[...]
    liota = jax.lax.broadcasted_iota(jnp.int32, (ROWS_PER_CORE, L), 1)
    fp = jnp.min(jnp.where(fwdv == PAD, liota, L), axis=1,
                 keepdims=True)
    unk = (liota < fp) & (fwdv == UNK)
    bitsf = unk.astype(jnp.float32)

    riota = jax.lax.broadcasted_iota(jnp.int32, (L, 128), 0)
    ciota = jax.lax.broadcasted_iota(jnp.int32, (L, 128), 1)
    r3 = riota >> 3
    w = jnp.exp2((riota & 7).astype(jnp.float32))
    pack_m = (jnp.where(r3 == ciota, w, 0.0)
              + jnp.where(r3 == ciota - 64, 1.0, 0.0))
    cmb = jnp.dot(bitsf, pack_m,
                  preferred_element_type=jnp.float32).astype(jnp.int32)

    nch = jnp.minimum((fp >> 3) + 1, NCHUNK)
    xb_vmem[:, 0:128] = cmb
    xb_vmem[:, 128:256] = jnp.broadcast_to(nch, (ROWS_PER_CORE, 128))
    pltpu.make_async_copy(xb_vmem, xb_sm, sem3).start()

    sub_iota = jax.lax.broadcasted_iota(jnp.int32, (8, 128), 0)
    lane_iota = jax.lax.broadcasted_iota(jnp.int32, (8, 128), 1)
    row_iota = jax.lax.broadcasted_iota(jnp.int32, (8, 1), 0)

    pltpu.make_async_copy(
        slab.at[pl.ds(0, 8)], slab.at[pl.ds(0, 8)], sem).wait()
    fbmat = jnp.zeros((8, 128), jnp.float32)
    for j in range(ROWS_PER_CORE):
        chunk = slab[j]
        rv = jnp.sum(jnp.where(sub_iota == (msub[j] & 7), chunk, 0.0),
                     axis=0, keepdims=True)
        fbmat = fbmat + jnp.where(row_iota == j, rv, 0.0)
    fbp = fbmat[:, END:END + 1]

    pltpu.make_async_copy(xa_vmem, xa_sm, sem2).wait()
    pltpu.make_async_copy(xb_vmem, xb_sm, sem3).wait()

    cnt = jnp.int32(ROWS_PER_CORE)
    for j in range(ROWS_PER_CORE):
        def chunk_body(k, cnt, j=j):
            bits = xb_sm[j, k]
            pops = xb_sm[j, 64 + k]

            @pl.when(bits != 0)
            def _chunk():
                b = base_b + j
                tl = pl.multiple_of(k << 3, 8)

                def hit_body(_, st, b=b, tl=tl, j=j):
                    bits_c, slot = st
                    low = bits_c & -bits_c
                    i = (((low & 0xAA) != 0).astype(jnp.int32)
                         + 2 * ((low & 0xCC) != 0).astype(jnp.int32)
                         + 4 * ((low & 0xF0) != 0).astype(jnp.int32))
                    t = xa_sm[j, tl + i]
                    cb = pl.multiple_of((t >> 7) << 7, 128)
                    pltpu.make_async_copy(
                        logits_ref.at[pl.ds(b, 1), pl.ds(tl, 8),
                                      pl.ds(cb, 128)],
                        slab.at[pl.ds(slot, 1)], sem).start()
                    msub[slot] = i | ((t & 127) << 3) | (j << 10)
                    return bits_c ^ low, slot + 1

                jax.lax.fori_loop(0, pops, hit_body, (bits, cnt))

            return cnt + pops

        cnt = jax.lax.fori_loop(0, xb_sm[j, 128], chunk_body, cnt)
    cnt_fin = cnt

    def wait8_body(_, carry):
        pltpu.make_async_copy(
            slab.at[pl.ds(0, 8)], slab.at[pl.ds(0, 8)], sem).wait()
        return carry

    def wait1_body(_, carry):
        pltpu.make_async_copy(
            slab.at[pl.ds(0, 1)], slab.at[pl.ds(0, 1)], sem).wait()
        return carry

    ncnt = cnt_fin - ROWS_PER_CORE
    jax.lax.fori_loop(0, ncnt >> 3, wait8_body, 0)
    jax.lax.fori_loop(0, ncnt & 7, wait1_body, 0)

    def ext_body(k, carry):
        acc, cntm = carry
        chunk = slab[k]
        m = msub[k]
        rv = jnp.sum(jnp.where(sub_iota == (m & 7), chunk, 0.0),
                     axis=0, keepdims=True)
        lm = lane_iota[0:1, :] == ((m >> 3) & 127)
        rowmask = row_iota == (m >> 10)
        hit = rowmask & lm
        nll = -jnp.log(rv)
        acc = acc + jnp.where(hit, nll, 0.0)
        cntm = cntm + jnp.where(hit, 1.0, 0.0)
        return acc, cntm

    def ext_pair(p, carry):
        k = ROWS_PER_CORE + p * 2
        return ext_body(k + 1, ext_body(k, carry))

    acc0 = jnp.zeros((8, 128), jnp.float32)
    npairs = (cnt_fin - ROWS_PER_CORE) >> 1
    acc, cntm = jax.lax.fori_loop(0, npairs, ext_pair, (acc0, acc0))
    acc, cntm = jax.lax.fori_loop(ROWS_PER_CORE + npairs * 2, cnt_fin,
                                  ext_body, (acc, cntm))

    ssum = jnp.sum(acc, axis=1, keepdims=True)
    cnt_v = jnp.sum(cntm, axis=1, keepdims=True)
    smean = ssum / jnp.maximum(cnt_v, 1.0)
    sent = jnp.where(cnt_v > 0, smean, -jnp.log(fbp))

    active = jnp.zeros((8, 1), jnp.float32)
    for j in range(ROWS_PER_CORE):
        b = base_b + j
        a = (ins_sm[b] < seq_sm[b]).astype(jnp.float32)
        active = active + jnp.where(row_iota == j, a, 0.0)

    num = jnp.sum(sent * active)
    den = jnp.sum(active)
    li = jax.lax.broadcasted_iota(jnp.int32, (1, 128), 1)
    out_ref[0] = jnp.where(li == 0, num, jnp.where(li == 1, den, 0.0))


def kernel(logits, forwarded_trgs, targets, sequence_lengths, inserted):
    fwd = forwarded_trgs.astype(jnp.int32)
    tgt = targets.astype(jnp.int32)
    seq = sequence_lengths.astype(jnp.int32)
    ins = inserted.astype(jnp.int32)

    out = pl.pallas_call(
        _ti_loss_kernel,
        grid_spec=pltpu.PrefetchScalarGridSpec(
            num_scalar_prefetch=2,
            grid=(2,),
            in_specs=[
                pl.BlockSpec((ROWS_PER_CORE, L), lambda i, *_: (i, 0)),
                pl.BlockSpec((ROWS_PER_CORE, L), lambda i, *_: (i, 0)),
                pl.BlockSpec(memory_space=pl.ANY),
            ],
            out_specs=pl.BlockSpec((1, 1, 128), lambda i, *_: (i, 0, 0)),
            scratch_shapes=[
                pltpu.VMEM((NSLOT, 8, 128), jnp.float32),
                pltpu.SMEM((NSLOT,), jnp.int32),
                pltpu.VMEM((ROWS_PER_CORE, L), jnp.int32),
                pltpu.SMEM((ROWS_PER_CORE, L), jnp.int32),
                pltpu.VMEM((ROWS_PER_CORE, XB_W), jnp.int32),
                pltpu.SMEM((ROWS_PER_CORE, XB_W), jnp.int32),
                pltpu.SemaphoreType.DMA,
                pltpu.SemaphoreType.DMA,
                pltpu.SemaphoreType.DMA,
            ],
        ),
        out_shape=jax.ShapeDtypeStruct((2, 1, 128), jnp.float32),
        compiler_params=pltpu.CompilerParams(
            dimension_semantics=("parallel",),
            vmem_limit_bytes=56 * 1024 * 1024,
        ),
    )(seq, ins, fwd, tgt, logits)

    num = out[0, 0, 0] + out[1, 0, 0]
    den = jnp.maximum(out[0, 0, 1] + out[1, 0, 1], 1.0)
    return num / den
